# baseline (speedup 1.0000x reference)
import functools

import jax
import jax.numpy as jnp
from jax import lax
from jax.experimental import pallas as pl
from jax.experimental.pallas import tpu as pltpu
from jax.experimental.pallas import tpu_sc as plsc

BATCH = 16384
DIM = 32
LANES = 16
CHUNK = 128

_info = plsc.get_sparse_core_info()
NC = _info.num_cores
NS = _info.num_subcores
NW = NC * NS
B_PER_W = BATCH // NW
NCHUNK = B_PER_W // CHUNK


def _bpr_body(u_hbm, i_hbm, j_hbm, ut_hbm, it_hbm, pos_hbm, neg_hbm,
              idx_u, idx_i, idx_j, rows_u, rows_i, rows_j,
              pos_v, neg_v, sem):
    wid = lax.axis_index("s") * NC + lax.axis_index("c")
    base = wid * B_PER_W

    for c in range(NCHUNK):
        off = base + c * CHUNK
        pltpu.sync_copy(u_hbm.at[pl.ds(off, CHUNK)], idx_u.at[c])
        pltpu.sync_copy(i_hbm.at[pl.ds(off, CHUNK)], idx_i.at[c])
        pltpu.sync_copy(j_hbm.at[pl.ds(off, CHUNK)], idx_j.at[c])

    copies = []
    for c in range(NCHUNK):
        dst = pl.ds(c * CHUNK, CHUNK)
        copies.append(pltpu.async_copy(ut_hbm.at[idx_u.at[c]], rows_u.at[dst], sem))
        copies.append(pltpu.async_copy(it_hbm.at[idx_i.at[c]], rows_i.at[dst], sem))
        copies.append(pltpu.async_copy(it_hbm.at[idx_j.at[c]], rows_j.at[dst], sem))
    for cp in copies:
        cp.wait()

    lanes_iota = lax.iota(jnp.int32, LANES)

    def lanesum(v):
        for sh in (8, 4, 2, 1):
            v = v + v.at[lanes_iota ^ sh].get(mode="promise_in_bounds")
        return v

    def group_body(g, carry):
        posacc = jnp.zeros((LANES,), jnp.float32)
        negacc = jnp.zeros((LANES,), jnp.float32)
        for k in range(LANES):
            r = g * LANES + k
            u0 = rows_u[r, pl.ds(0, LANES)]
            u1 = rows_u[r, pl.ds(LANES, LANES)]
            i0 = rows_i[r, pl.ds(0, LANES)]
            i1 = rows_i[r, pl.ds(LANES, LANES)]
            j0 = rows_j[r, pl.ds(0, LANES)]
            j1 = rows_j[r, pl.ds(LANES, LANES)]
            ps = lanesum(u0 * i0 + u1 * i1)
            ns = lanesum(u0 * j0 + u1 * j1)
            posacc = jnp.where(lanes_iota == k, ps, posacc)
            negacc = jnp.where(lanes_iota == k, ns, negacc)
        pos_v[pl.ds(g * LANES, LANES)] = posacc
        neg_v[pl.ds(g * LANES, LANES)] = negacc
        return carry

    lax.fori_loop(0, B_PER_W // LANES, group_body, 0)

    pltpu.sync_copy(pos_v, pos_hbm.at[pl.ds(base, B_PER_W)])
    pltpu.sync_copy(neg_v, neg_hbm.at[pl.ds(base, B_PER_W)])


@jax.jit
def _bpr_call(u, i, j, user_table, item_table):
    mesh = plsc.VectorSubcoreMesh(core_axis_name="c", subcore_axis_name="s")
    f = functools.partial(
        pl.kernel,
        mesh=mesh,
        compiler_params=pltpu.CompilerParams(use_tc_tiling_on_sc=False),
        out_type=[
            jax.ShapeDtypeStruct((BATCH,), jnp.float32),
            jax.ShapeDtypeStruct((BATCH,), jnp.float32),
        ],
        scratch_types=[
            pltpu.VMEM((NCHUNK, CHUNK), jnp.int32),
            pltpu.VMEM((NCHUNK, CHUNK), jnp.int32),
            pltpu.VMEM((NCHUNK, CHUNK), jnp.int32),
            pltpu.VMEM((B_PER_W, DIM), jnp.float32),
            pltpu.VMEM((B_PER_W, DIM), jnp.float32),
            pltpu.VMEM((B_PER_W, DIM), jnp.float32),
            pltpu.VMEM((B_PER_W,), jnp.float32),
            pltpu.VMEM((B_PER_W,), jnp.float32),
            pltpu.SemaphoreType.DMA,
        ],
    )(_bpr_body)
    return f(u, i, j, user_table, item_table)


def kernel(u, i, j, user_table, item_table):
    u = u.astype(jnp.int32)
    i = i.astype(jnp.int32)
    j = j.astype(jnp.int32)
    pos, neg = _bpr_call(u, i, j, user_table, item_table)
    return (pos, neg)


# speedup vs baseline: 1.1764x; 1.1764x over previous
import functools

import jax
import jax.numpy as jnp
from jax import lax
from jax.experimental import pallas as pl
from jax.experimental.pallas import tpu as pltpu
from jax.experimental.pallas import tpu_sc as plsc

BATCH = 16384
DIM = 32
LANES = 16

_info = plsc.get_sparse_core_info()
NC = _info.num_cores
NS = _info.num_subcores
NW = NC * NS
B_PER_W = BATCH // NW
NGROUP = B_PER_W // LANES
RING = 4
NITER = NGROUP // RING
IDX_PAD = (NGROUP + RING) * LANES


def _bpr_body(u_hbm, i_hbm, j_hbm, ut_hbm, it_hbm, pos_hbm, neg_hbm,
              idx_u, idx_i, idx_j, rows_u, rows_i, rows_j,
              pos_v, neg_v, sem0, sem1, sem2, sem3):
    sems = (sem0, sem1, sem2, sem3)
    wid = lax.axis_index("s") * NC + lax.axis_index("c")
    base = wid * B_PER_W

    pltpu.sync_copy(u_hbm.at[pl.ds(base, B_PER_W)], idx_u.at[pl.ds(0, B_PER_W)])
    pltpu.sync_copy(i_hbm.at[pl.ds(base, B_PER_W)], idx_i.at[pl.ds(0, B_PER_W)])
    pltpu.sync_copy(j_hbm.at[pl.ds(base, B_PER_W)], idx_j.at[pl.ds(0, B_PER_W)])
    zeros16 = jnp.zeros((LANES,), jnp.int32)
    for k in range(RING):
        off = (NGROUP + k) * LANES
        idx_u[pl.ds(off, LANES)] = zeros16
        idx_i[pl.ds(off, LANES)] = zeros16
        idx_j[pl.ds(off, LANES)] = zeros16

    def fire_group(gg, slot):
        sem = sems[slot]
        vu = idx_u[pl.ds(gg * LANES, LANES)]
        vi = idx_i[pl.ds(gg * LANES, LANES)]
        vj = idx_j[pl.ds(gg * LANES, LANES)]
        for k in range(LANES):
            dk = pl.ds(k, 1)
            pltpu.async_copy(ut_hbm.at[pl.ds(vu[k], 1)], rows_u.at[slot, dk], sem)
            pltpu.async_copy(it_hbm.at[pl.ds(vi[k], 1)], rows_i.at[slot, dk], sem)
            pltpu.async_copy(it_hbm.at[pl.ds(vj[k], 1)], rows_j.at[slot, dk], sem)

    def drain_group(slot):
        sem = sems[slot]
        src = ut_hbm.at[pl.ds(0, 1)]
        for k in range(LANES):
            dk = pl.ds(k, 1)
            pltpu.make_async_copy(src, rows_u.at[slot, dk], sem).wait()
            pltpu.make_async_copy(src, rows_i.at[slot, dk], sem).wait()
            pltpu.make_async_copy(src, rows_j.at[slot, dk], sem).wait()

    lanes_iota = lax.iota(jnp.int32, LANES)

    def lanesum(v):
        for sh in (8, 4, 2, 1):
            v = v + v.at[lanes_iota ^ sh].get(mode="promise_in_bounds")
        return v

    def compute_group(gg, slot):
        posacc = jnp.zeros((LANES,), jnp.float32)
        negacc = jnp.zeros((LANES,), jnp.float32)
        for k in range(LANES):
            u0 = rows_u[slot, k, pl.ds(0, LANES)]
            u1 = rows_u[slot, k, pl.ds(LANES, LANES)]
            i0 = rows_i[slot, k, pl.ds(0, LANES)]
            i1 = rows_i[slot, k, pl.ds(LANES, LANES)]
            j0 = rows_j[slot, k, pl.ds(0, LANES)]
            j1 = rows_j[slot, k, pl.ds(LANES, LANES)]
            ps = lanesum(u0 * i0 + u1 * i1)
            ns = lanesum(u0 * j0 + u1 * j1)
            posacc = jnp.where(lanes_iota == k, ps, posacc)
            negacc = jnp.where(lanes_iota == k, ns, negacc)
        pos_v[pl.ds(gg * LANES, LANES)] = posacc
        neg_v[pl.ds(gg * LANES, LANES)] = negacc

    for s in range(RING):
        fire_group(s, s)

    def ring_body(h, carry):
        for s in range(RING):
            gg = h * RING + s
            drain_group(s)
            compute_group(gg, s)
            fire_group(gg + RING, s)
        return carry

    lax.fori_loop(0, NITER, ring_body, 0)

    for s in range(RING):
        drain_group(s)

    pltpu.sync_copy(pos_v, pos_hbm.at[pl.ds(base, B_PER_W)])
    pltpu.sync_copy(neg_v, neg_hbm.at[pl.ds(base, B_PER_W)])


@jax.jit
def _bpr_call(u, i, j, user_table, item_table):
    mesh = plsc.VectorSubcoreMesh(core_axis_name="c", subcore_axis_name="s")
    f = functools.partial(
        pl.kernel,
        mesh=mesh,
        out_type=[
            jax.ShapeDtypeStruct((BATCH,), jnp.float32),
            jax.ShapeDtypeStruct((BATCH,), jnp.float32),
        ],
        scratch_types=[
            pltpu.VMEM((IDX_PAD,), jnp.int32),
            pltpu.VMEM((IDX_PAD,), jnp.int32),
            pltpu.VMEM((IDX_PAD,), jnp.int32),
            pltpu.VMEM((RING, LANES, DIM), jnp.float32),
            pltpu.VMEM((RING, LANES, DIM), jnp.float32),
            pltpu.VMEM((RING, LANES, DIM), jnp.float32),
            pltpu.VMEM((B_PER_W,), jnp.float32),
            pltpu.VMEM((B_PER_W,), jnp.float32),
            pltpu.SemaphoreType.DMA,
            pltpu.SemaphoreType.DMA,
            pltpu.SemaphoreType.DMA,
            pltpu.SemaphoreType.DMA,
        ],
    )(_bpr_body)
    return f(u, i, j, user_table, item_table)


def kernel(u, i, j, user_table, item_table):
    u = u.astype(jnp.int32)
    i = i.astype(jnp.int32)
    j = j.astype(jnp.int32)
    pos, neg = _bpr_call(u, i, j, user_table, item_table)
    return (pos, neg)


# speedup vs baseline: 2.6973x; 2.2929x over previous
import functools

import jax
import jax.numpy as jnp
from jax import lax
from jax.experimental import pallas as pl
from jax.experimental.pallas import tpu as pltpu
from jax.experimental.pallas import tpu_sc as plsc

BATCH = 16384
DIM = 32
LANES = 16
WIN = 128
PHASE = 8

_info = plsc.get_sparse_core_info()
NC = _info.num_cores
NS = _info.num_subcores
NW = NC * NS
B_PER_W = BATCH // NW
NGROUP = B_PER_W // LANES


def _bpr_body(u_hbm, i_hbm, j_hbm, ut_hbm, it_hbm, pos_hbm, neg_hbm,
              idx_u, idx_i, idx_j, wu, wi, wj, pos_v, neg_v, sem):
    wid = lax.axis_index("s") * NC + lax.axis_index("c")
    base = wid * B_PER_W

    pltpu.sync_copy(u_hbm.at[pl.ds(base, B_PER_W)], idx_u)
    pltpu.sync_copy(i_hbm.at[pl.ds(base, B_PER_W)], idx_i)
    pltpu.sync_copy(j_hbm.at[pl.ds(base, B_PER_W)], idx_j)

    lanes = lax.iota(jnp.int32, LANES)
    slot = lanes & (PHASE - 1)

    def fire_phase(vu, vi, vj, ph):
        for t in range(PHASE):
            k = ph * PHASE + t
            ou = pl.multiple_of((vu[k] >> 7) * WIN, WIN)
            oi = pl.multiple_of((vi[k] >> 7) * WIN, WIN)
            oj = pl.multiple_of((vj[k] >> 7) * WIN, WIN)
            pltpu.async_copy(ut_hbm.at[:, pl.ds(ou, WIN)], wu.at[t], sem)
            pltpu.async_copy(it_hbm.at[:, pl.ds(oi, WIN)], wi.at[t], sem)
            pltpu.async_copy(it_hbm.at[:, pl.ds(oj, WIN)], wj.at[t], sem)

    def drain_phase():
        src = ut_hbm.at[:, pl.ds(0, WIN)]
        for t in range(PHASE):
            pltpu.make_async_copy(src, wu.at[t], sem).wait()
            pltpu.make_async_copy(src, wi.at[t], sem).wait()
            pltpu.make_async_copy(src, wj.at[t], sem).wait()

    def extract_phase(rl_u, rl_i, rl_j, ph):
        perm = ph * PHASE + slot
        ru = rl_u.at[perm].get(mode="promise_in_bounds")
        ri = rl_i.at[perm].get(mode="promise_in_bounds")
        rj = rl_j.at[perm].get(mode="promise_in_bounds")
        accp = jnp.zeros((LANES,), jnp.float32)
        accn = jnp.zeros((LANES,), jnp.float32)
        for c in range(DIM):
            cvec = jnp.full((LANES,), c, jnp.int32)
            gu = plsc.load_gather(wu, [slot, cvec, ru])
            gi = plsc.load_gather(wi, [slot, cvec, ri])
            gj = plsc.load_gather(wj, [slot, cvec, rj])
            accp = accp + gu * gi
            accn = accn + gu * gj
        return accp, accn

    def group_body(g, carry):
        goff = g * LANES
        vu = idx_u[pl.ds(goff, LANES)]
        vi = idx_i[pl.ds(goff, LANES)]
        vj = idx_j[pl.ds(goff, LANES)]
        rl_u = vu & (WIN - 1)
        rl_i = vi & (WIN - 1)
        rl_j = vj & (WIN - 1)

        fire_phase(vu, vi, vj, 0)
        drain_phase()
        p0, n0 = extract_phase(rl_u, rl_i, rl_j, 0)
        fire_phase(vu, vi, vj, 1)
        drain_phase()
        p1, n1 = extract_phase(rl_u, rl_i, rl_j, 1)

        lo = lanes < PHASE
        pos_v[pl.ds(goff, LANES)] = jnp.where(lo, p0, p1)
        neg_v[pl.ds(goff, LANES)] = jnp.where(lo, n0, n1)
        return carry

    lax.fori_loop(0, NGROUP, group_body, 0)

    pltpu.sync_copy(pos_v, pos_hbm.at[pl.ds(base, B_PER_W)])
    pltpu.sync_copy(neg_v, neg_hbm.at[pl.ds(base, B_PER_W)])


@jax.jit
def _bpr_call(u, i, j, user_table, item_table):
    ut_t = user_table.T
    it_t = item_table.T
    mesh = plsc.VectorSubcoreMesh(core_axis_name="c", subcore_axis_name="s")
    f = functools.partial(
        pl.kernel,
        mesh=mesh,
        compiler_params=pltpu.CompilerParams(needs_layout_passes=False),
        out_type=[
            jax.ShapeDtypeStruct((BATCH,), jnp.float32),
            jax.ShapeDtypeStruct((BATCH,), jnp.float32),
        ],
        scratch_types=[
            pltpu.VMEM((B_PER_W,), jnp.int32),
            pltpu.VMEM((B_PER_W,), jnp.int32),
            pltpu.VMEM((B_PER_W,), jnp.int32),
            pltpu.VMEM((PHASE, DIM, WIN), jnp.float32),
            pltpu.VMEM((PHASE, DIM, WIN), jnp.float32),
            pltpu.VMEM((PHASE, DIM, WIN), jnp.float32),
            pltpu.VMEM((B_PER_W,), jnp.float32),
            pltpu.VMEM((B_PER_W,), jnp.float32),
            pltpu.SemaphoreType.DMA,
        ],
    )(_bpr_body)
    return f(u, i, j, ut_t, it_t)


def kernel(u, i, j, user_table, item_table):
    u = u.astype(jnp.int32)
    i = i.astype(jnp.int32)
    j = j.astype(jnp.int32)
    pos, neg = _bpr_call(u, i, j, user_table, item_table)
    return (pos, neg)
